# Initial kernel scaffold; baseline (speedup 1.0000x reference)
import jax
import jax.numpy as jnp
from jax.experimental import pallas as pl


def kernel(x, segment_tokens, token_table, segment_table, pe):
    raise NotImplementedError("write your pallas kernel here")


# speedup vs baseline: 3.1772x; 3.1772x over previous
import functools

import jax
import jax.numpy as jnp
from jax import lax
from jax.experimental import pallas as pl
from jax.experimental.pallas import tpu as pltpu
from jax.experimental.pallas import tpu_sc as plsc

_LANES = 16
_KTOK = 128


@functools.partial(jax.jit, static_argnums=(4, 5, 6))
def _sc_embed(x2, c2, token_table, comb, T, D, NW):
  G = (T // _KTOK) // NW
  mesh = plsc.VectorSubcoreMesh(core_axis_name="c", subcore_axis_name="s")

  @functools.partial(
      pl.kernel,
      mesh=mesh,
      out_type=jax.ShapeDtypeStruct((T, D), jnp.float32),
      scratch_types=[
          pltpu.VMEM((_KTOK,), jnp.int32),
          pltpu.VMEM((_KTOK,), jnp.int32),
          pltpu.VMEM((_KTOK, D), jnp.float32),
          pltpu.VMEM((_KTOK, D), jnp.float32),
          pltpu.SemaphoreType.DMA,
          pltpu.SemaphoreType.DMA,
      ],
  )
  def k(x_hbm, c_hbm, tab_hbm, comb_hbm, out_hbm, xi_v, ci_v, rows_v, crows_v,
        sem_a, sem_b):
    wid = lax.axis_index("s") * 2 + lax.axis_index("c")
    row0 = wid * G

    def body(g, carry):
      r = row0 + g
      pltpu.sync_copy(x_hbm.at[r], xi_v)
      pltpu.sync_copy(c_hbm.at[r], ci_v)
      cp_a = pltpu.async_copy(tab_hbm.at[xi_v], rows_v, sem_a)
      cp_b = pltpu.async_copy(comb_hbm.at[ci_v], crows_v, sem_b)
      cp_a.wait()
      cp_b.wait()

      def add_body(i, c2_):
        for j in range(D // _LANES):
          plsc.addupdate(rows_v.at[i, pl.ds(j * _LANES, _LANES)],
                         crows_v[i, pl.ds(j * _LANES, _LANES)])
        return c2_

      lax.fori_loop(0, _KTOK, add_body, 0, unroll=2)
      pltpu.sync_copy(rows_v, out_hbm.at[pl.ds(r * _KTOK, _KTOK)])
      return carry

    lax.fori_loop(0, G, body, 0)

  return k(x2, c2, token_table, comb)


def kernel(x, segment_tokens, token_table, segment_table, pe):
  B, L = x.shape
  V, D = token_table.shape
  T = B * L
  NW = 32
  comb = (segment_table.astype(jnp.float32)[:, None, :]
          + pe[:L, :][None, :, :]).reshape(2 * L, D)
  cidx = (segment_tokens.astype(jnp.int32) * L
          + jnp.arange(L, dtype=jnp.int32)[None, :])
  x2 = x.astype(jnp.int32).reshape(T // _KTOK, _KTOK)
  c2 = cidx.reshape(T // _KTOK, _KTOK)
  out = _sc_embed(x2, c2, token_table, comb, T, D, NW)
  return out.reshape(B, L, D)


# speedup vs baseline: 4.3284x; 1.3623x over previous
import functools

import jax
import jax.numpy as jnp
from jax import lax
from jax.experimental import pallas as pl
from jax.experimental.pallas import tpu as pltpu
from jax.experimental.pallas import tpu_sc as plsc

_LANES = 16
_KTOK = 128


@functools.partial(jax.jit, static_argnums=(3, 4, 5))
def _sc_embed(idx2, token_table, comb, T, D, NW):
  G = (T // _KTOK) // NW
  mesh = plsc.VectorSubcoreMesh(core_axis_name="c", subcore_axis_name="s")

  @functools.partial(
      pl.kernel,
      mesh=mesh,
      out_type=jax.ShapeDtypeStruct((T, D), jnp.float32),
      scratch_types=[
          pltpu.VMEM((2, 2, _KTOK), jnp.int32),
          pltpu.VMEM((2, _KTOK, D), jnp.float32),
          pltpu.VMEM((2, _KTOK, D), jnp.float32),
      ] + [pltpu.SemaphoreType.DMA] * 8,
  )
  def k(idx_hbm, tab_hbm, comb_hbm, out_hbm, idx_v, rows_v, crows_v,
        s_i0, s_i1, s_ga0, s_ga1, s_gb0, s_gb1, s_o0, s_o1):
    sem_i = (s_i0, s_i1)
    sem_ga = (s_ga0, s_ga1)
    sem_gb = (s_gb0, s_gb1)
    sem_o = (s_o0, s_o1)
    wid = lax.axis_index("s") * 2 + lax.axis_index("c")
    row0 = wid * G

    def issue_idx(r, p):
      pltpu.async_copy(idx_hbm.at[r], idx_v.at[p], sem_i[p])

    def wait_idx(p):
      pltpu.make_async_copy(idx_hbm.at[0], idx_v.at[p], sem_i[p]).wait()

    def issue_gath(p):
      pltpu.async_copy(tab_hbm.at[idx_v.at[p, 0]], rows_v.at[p], sem_ga[p])
      pltpu.async_copy(comb_hbm.at[idx_v.at[p, 1]], crows_v.at[p], sem_gb[p])

    def wait_gath(p):
      pltpu.make_async_copy(tab_hbm.at[idx_v.at[p, 0]], rows_v.at[p],
                            sem_ga[p]).wait()
      pltpu.make_async_copy(comb_hbm.at[idx_v.at[p, 1]], crows_v.at[p],
                            sem_gb[p]).wait()

    def issue_out(r, p):
      pltpu.async_copy(rows_v.at[p], out_hbm.at[pl.ds(r * _KTOK, _KTOK)],
                       sem_o[p])

    def wait_out(p):
      pltpu.make_async_copy(rows_v.at[p], out_hbm.at[pl.ds(0, _KTOK)],
                            sem_o[p]).wait()

    def compute(p):
      def add_body(i, c_):
        for j in range(D // _LANES):
          plsc.addupdate(rows_v.at[p, i, pl.ds(j * _LANES, _LANES)],
                         crows_v[p, i, pl.ds(j * _LANES, _LANES)])
        return c_

      lax.fori_loop(0, _KTOK, add_body, 0, unroll=4)

    issue_idx(row0, 0)
    issue_idx(row0 + 1, 1)
    wait_idx(0)
    issue_gath(0)

    def body(t, carry):
      for b in range(2):
        g = 2 * t + b
        p = b
        q = 1 - b
        r = row0 + g
        wait_gath(p)
        if b == 0:
          wait_idx(q)

          @pl.when(t >= 1)
          def _():
            wait_out(q)

          issue_gath(q)

          @pl.when(t < G // 2 - 1)
          def _():
            issue_idx(r + 2, p)
        else:
          @pl.when(t < G // 2 - 1)
          def _():
            wait_idx(q)
            wait_out(q)
            issue_gath(q)
            issue_idx(r + 2, p)
        compute(p)
        issue_out(r, p)
      return carry

    lax.fori_loop(0, G // 2, body, 0)
    wait_out(0)
    wait_out(1)

  return k(idx2, token_table, comb)


def kernel(x, segment_tokens, token_table, segment_table, pe):
  B, L = x.shape
  V, D = token_table.shape
  T = B * L
  NW = 32
  comb = (segment_table.astype(jnp.float32)[:, None, :]
          + pe[:L, :][None, :, :]).reshape(2 * L, D)
  cidx = (segment_tokens.astype(jnp.int32) * L
          + jnp.arange(L, dtype=jnp.int32)[None, :])
  x2 = x.astype(jnp.int32).reshape(T // _KTOK, _KTOK)
  c2 = cidx.reshape(T // _KTOK, _KTOK)
  idx2 = jnp.stack([x2, c2], axis=1)
  out = _sc_embed(idx2, token_table, comb, T, D, NW)
  return out.reshape(B, L, D)
